# Initial kernel scaffold; baseline (speedup 1.0000x reference)
import jax
import jax.numpy as jnp
from jax.experimental import pallas as pl


def kernel(X, embedding_table):
    raise NotImplementedError("write your pallas kernel here")


# speedup vs baseline: 1.9782x; 1.9782x over previous
import functools

import jax
import jax.numpy as jnp
from jax import lax
from jax.experimental import pallas as pl
from jax.experimental.pallas import tpu as pltpu
from jax.experimental.pallas import tpu_sc as plsc

VOCAB = 8192
NTOK = 4 * 2048

_NC, _NS = 2, 16
_NW = _NC * _NS
_TPW = NTOK // _NW
_CB = 4
_NCHUNK = _TPW // _CB
_NBUF = 2


def _gather_body(idx_hbm, table_hbm, out_hbm, idx_v, bufs, gsems, ssems):
    wid = lax.axis_index("s") * _NC + lax.axis_index("c")
    row0 = wid * _TPW

    pltpu.sync_copy(idx_hbm.at[wid], idx_v)

    def start_gather(b, c):
        pltpu.async_copy(table_hbm.at[idx_v.at[c]], bufs[b], gsems[b])

    def wait_gather(b, c):
        pltpu.make_async_copy(table_hbm.at[idx_v.at[c]], bufs[b], gsems[b]).wait()

    def start_scatter(b, c):
        pltpu.async_copy(bufs[b], out_hbm.at[pl.ds(row0 + c * _CB, _CB)], ssems[b])

    def wait_scatter(b, c):
        pltpu.make_async_copy(
            bufs[b], out_hbm.at[pl.ds(row0 + c * _CB, _CB)], ssems[b]
        ).wait()

    for b in range(_NBUF):
        start_gather(b, b)

    def group(g, _):
        for b in range(_NBUF):
            c = g * _NBUF + b
            wait_gather(b, c)
            start_scatter(b, c)
            nc = c + _NBUF

            @pl.when(nc < _NCHUNK)
            def _():
                wait_scatter(b, c)
                start_gather(b, nc)

        return _

    lax.fori_loop(0, _NCHUNK // _NBUF, group, None)

    for b in range(_NBUF):
        wait_scatter(b, _NCHUNK - _NBUF + b)


@jax.jit
def _sc_gather(idx, table):
    mesh = plsc.VectorSubcoreMesh(core_axis_name="c", subcore_axis_name="s")
    scratch = (
        pltpu.VMEM((_NCHUNK, _CB), jnp.int32),
        tuple(pltpu.VMEM((_CB, VOCAB), jnp.float32) for _ in range(_NBUF)),
        tuple(pltpu.SemaphoreType.DMA for _ in range(_NBUF)),
        tuple(pltpu.SemaphoreType.DMA for _ in range(_NBUF)),
    )
    return pl.kernel(
        _gather_body,
        out_type=jax.ShapeDtypeStruct((NTOK, VOCAB), jnp.float32),
        mesh=mesh,
        scratch_types=scratch,
    )(idx, table)


def kernel(X, embedding_table):
    B, S = X.shape
    idx = X.astype(jnp.int32).reshape(_NW, _NCHUNK, _CB)
    flat = _sc_gather(idx, embedding_table)
    return flat.reshape(B, S, VOCAB)
